# baseline (speedup 1.0000x reference)
import functools

import jax
import jax.numpy as jnp
from jax import lax
from jax.experimental import pallas as pl
from jax.experimental.pallas import tpu as pltpu
from jax.experimental.pallas import tpu_sc as plsc

B, S = 1024, 200
N = B * S
D = 64
HD = 128

NC, NS = 2, 16
NW = NC * NS
NSLICE = 4
NH = N // NSLICE
PER_W = NH // NW
CHUNK = 400
STEPS = PER_W // CHUNK
LANES = 16


def _proj_body(t_ref, w_ref, o_ref):
    o_ref[...] = jnp.dot(t_ref[...], w_ref[...],
                         preferred_element_type=jnp.float32)


def _proj_body_bf16(t_ref, w_ref, o_ref):
    o_ref[...] = jnp.dot(t_ref[...], w_ref[...].astype(jnp.bfloat16),
                         preferred_element_type=jnp.float32)


def _project(table, wblk, rows_blk=2048, bf16=False):
    v = table.shape[0]
    grid = (v + rows_blk - 1) // rows_blk
    return pl.pallas_call(
        _proj_body_bf16 if bf16 else _proj_body,
        grid=(grid,),
        in_specs=[
            pl.BlockSpec((rows_blk, D), lambda i: (i, 0)),
            pl.BlockSpec((D, HD), lambda i: (0, 0)),
        ],
        out_specs=pl.BlockSpec((rows_blk, HD), lambda i: (i, 0)),
        out_shape=jax.ShapeDtypeStruct((v, HD), jnp.float32),
    )(table, wblk)


def _cross_body(t_ref, wt_ref, i_ref, wi_ref, b_ref, o_ref):
    pt = jnp.dot(t_ref[...], wt_ref[...], preferred_element_type=jnp.float32)
    pi = jnp.dot(i_ref[...], wi_ref[...], preferred_element_type=jnp.float32)
    o_ref[...] = pt[:, None, :] + pi[None, :, :] + b_ref[...][None, :, :]


def _cross_table(emb_test, w_test, emb_inter, w_inter, b_proj):
    vt = emb_test.shape[0]
    out = pl.pallas_call(
        _cross_body,
        in_specs=[
            pl.BlockSpec((vt, D), lambda: (0, 0)),
            pl.BlockSpec((D, HD), lambda: (0, 0)),
            pl.BlockSpec((3, D), lambda: (0, 0)),
            pl.BlockSpec((D, HD), lambda: (0, 0)),
            pl.BlockSpec((1, HD), lambda: (0, 0)),
        ],
        out_specs=pl.BlockSpec((vt, 3, HD), lambda: (0, 0, 0)),
        out_shape=jax.ShapeDtypeStruct((vt, 3, HD), jnp.float32),
    )(emb_test, w_test, emb_inter, w_inter, b_proj.reshape(1, HD))
    return out.reshape(vt * 3, HD)


def _sc_gather_sum(p_ti, p_quest, p_tag, it, ii, iq, ig, half):
    mesh = plsc.VectorSubcoreMesh(core_axis_name="c", subcore_axis_name="s")
    idx_t = pltpu.VMEM((CHUNK,), jnp.int32)
    rows_t = pltpu.VMEM((CHUNK, HD), jnp.float32)
    sem = pltpu.SemaphoreType.DMA
    off = half * NH

    @functools.partial(
        pl.kernel,
        mesh=mesh,
        out_type=jax.ShapeDtypeStruct((NH, HD), jnp.float32),
        scratch_types=[idx_t] * 8 + [rows_t] * 2 + [sem] * 8,
    )
    def k(ti_h, quest_h, tag_h, it_h, ii_h, iq_h, ig_h, o_h,
          ivt0, ivt1, ivi0, ivi1, ivq0, ivq1, ivg0, ivg1, r0, r1,
          si0, si1, sg0, sg1, sa0, sa1, so0, so1):
        ivt, ivi, ivq, ivg = (ivt0, ivt1), (ivi0, ivi1), (ivq0, ivq1), (ivg0, ivg1)
        r = (r0, r1)
        s_idx, s_g, s_add, s_out = (si0, si1), (sg0, sg1), (sa0, sa1), (so0, so1)

        wid = lax.axis_index("s") * NC + lax.axis_index("c")
        w_base = wid * PER_W

        def base(i):
            return pl.multiple_of(w_base + i * CHUNK, CHUNK)

        def issue_idx(i):
            j = i % 2
            sl = pl.ds(off + base(i), CHUNK)
            return [
                pltpu.async_copy(it_h.at[sl], ivt[j], s_idx[j]),
                pltpu.async_copy(ii_h.at[sl], ivi[j], s_idx[j]),
                pltpu.async_copy(iq_h.at[sl], ivq[j], s_idx[j]),
                pltpu.async_copy(ig_h.at[sl], ivg[j], s_idx[j]),
            ]

        idx_d = {0: issue_idx(0)}
        add_d = {}
        out_d = {}
        for i in range(STEPS):
            j = i % 2
            for d in idx_d.pop(i):
                d.wait()

            def fuse(kk, carry, _j=j):
                sl = pl.ds(kk * LANES, LANES)
                ivt[_j][sl] = ivt[_j][sl] * 3 + ivi[_j][sl]
                return carry

            lax.fori_loop(0, CHUNK // LANES, fuse, 0)
            if i >= 2:
                out_d.pop(i - 2).wait()
            g = pltpu.async_copy(ti_h.at[ivt[j]], r[j], s_g[j])
            if i >= 1:
                for d in add_d.pop(i - 1):
                    d.wait()
                out_d[i - 1] = pltpu.async_copy(
                    r[1 - j], o_h.at[pl.ds(base(i - 1), CHUNK)], s_out[1 - j])
            if i + 1 < STEPS:
                idx_d[i + 1] = issue_idx(i + 1)
            g.wait()
            add_d[i] = [
                pltpu.async_copy(quest_h.at[ivq[j]], r[j], s_add[j], add=True),
                pltpu.async_copy(tag_h.at[ivg[j]], r[j], s_add[j], add=True),
            ]
        last = STEPS - 1
        jl = last % 2
        for d in add_d.pop(last):
            d.wait()
        out_d[last] = pltpu.async_copy(
            r[jl], o_h.at[pl.ds(base(last), CHUNK)], s_out[jl])
        out_d.pop(last - 1).wait()
        out_d.pop(last).wait()

    return k(p_ti, p_quest, p_tag, it, ii, iq, ig)


ROWS = 2048
GRID_H = NH // ROWS


def _ln_math(x, g, b):
    mu = jnp.mean(x, axis=-1, keepdims=True)
    xc = x - mu
    var = jnp.mean(xc * xc, axis=-1, keepdims=True)
    return xc * lax.rsqrt(var + 1e-6) * g + b


def _ln_body0(x_ref, g_ref, beta_ref, o_ref):
    o_ref[...] = _ln_math(x_ref[...], g_ref[...], beta_ref[...])


def _ln_body1(x_ref, g_ref, beta_ref, full_ref, o_ref):
    del full_ref
    o_ref[...] = _ln_math(x_ref[...], g_ref[...], beta_ref[...])


def _tc_epilogue(xs, ln_gamma, ln_beta):
    g2 = ln_gamma.reshape(1, HD)
    b2 = ln_beta.reshape(1, HD)
    y = pl.pallas_call(
        _ln_body0,
        grid=(GRID_H,),
        in_specs=[
            pl.BlockSpec((ROWS, HD), lambda i: (i, 0)),
            pl.BlockSpec((1, HD), lambda i: (0, 0)),
            pl.BlockSpec((1, HD), lambda i: (0, 0)),
        ],
        out_specs=pl.BlockSpec((ROWS, HD), lambda i: (i, 0)),
        out_shape=jax.ShapeDtypeStruct((N, HD), jnp.float32),
    )(xs[0], g2, b2)
    for h in range(1, NSLICE):
        y = pl.pallas_call(
            _ln_body1,
            grid=(GRID_H,),
            in_specs=[
                pl.BlockSpec((ROWS, HD), lambda i: (i, 0)),
                pl.BlockSpec((1, HD), lambda i: (0, 0)),
                pl.BlockSpec((1, HD), lambda i: (0, 0)),
                pl.BlockSpec(memory_space=pl.ANY),
            ],
            out_specs=pl.BlockSpec(
                (ROWS, HD), lambda i, _h=h: (i + _h * GRID_H, 0)),
            out_shape=jax.ShapeDtypeStruct((N, HD), jnp.float32),
            input_output_aliases={3: 0},
        )(xs[h], g2, b2, y)
    return y


def kernel(testId, assessmentItemID, KnowledgeTag, answerCode, mask,
           interaction, emb_interaction, emb_test, emb_question, emb_tag,
           W_proj, b_proj, ln_gamma, ln_beta):
    it = testId.reshape(N).astype(jnp.int32)
    ii = interaction.reshape(N).astype(jnp.int32)
    iq = assessmentItemID.reshape(N).astype(jnp.int32)
    ig = KnowledgeTag.reshape(N).astype(jnp.int32)
    p_ti = _cross_table(emb_test, W_proj[D:2 * D], emb_interaction,
                        W_proj[0:D], b_proj)
    p_quest = _project(emb_question.astype(jnp.bfloat16),
                       W_proj[2 * D:3 * D], bf16=True)
    p_tag = _project(emb_tag, W_proj[3 * D:4 * D])
    xs = [_sc_gather_sum(p_ti, p_quest, p_tag, it, ii, iq, ig, h)
          for h in range(NSLICE)]
    x = _tc_epilogue(xs, ln_gamma, ln_beta)
    return x.reshape(B, S, HD)


# speedup vs baseline: 1.0199x; 1.0199x over previous
import functools

import jax
import jax.numpy as jnp
from jax import lax
from jax.experimental import pallas as pl
from jax.experimental.pallas import tpu as pltpu
from jax.experimental.pallas import tpu_sc as plsc

B, S = 1024, 200
N = B * S
D = 64
HD = 128

NC, NS = 2, 16
NW = NC * NS
NSLICE = 2
NH = N // NSLICE
PER_W = NH // NW
CHUNK = 400
STEPS = PER_W // CHUNK
LANES = 16


def _proj_body(t_ref, w_ref, o_ref):
    o_ref[...] = jnp.dot(t_ref[...], w_ref[...],
                         preferred_element_type=jnp.float32)


def _proj_body_bf16(t_ref, w_ref, o_ref):
    o_ref[...] = jnp.dot(t_ref[...], w_ref[...].astype(jnp.bfloat16),
                         preferred_element_type=jnp.float32)


def _project(table, wblk, rows_blk=2048, bf16=False):
    v = table.shape[0]
    grid = (v + rows_blk - 1) // rows_blk
    return pl.pallas_call(
        _proj_body_bf16 if bf16 else _proj_body,
        grid=(grid,),
        in_specs=[
            pl.BlockSpec((rows_blk, D), lambda i: (i, 0)),
            pl.BlockSpec((D, HD), lambda i: (0, 0)),
        ],
        out_specs=pl.BlockSpec((rows_blk, HD), lambda i: (i, 0)),
        out_shape=jax.ShapeDtypeStruct((v, HD), jnp.float32),
    )(table, wblk)


def _cross_body(t_ref, wt_ref, i_ref, wi_ref, b_ref, o_ref):
    pt = jnp.dot(t_ref[...], wt_ref[...], preferred_element_type=jnp.float32)
    pi = jnp.dot(i_ref[...], wi_ref[...], preferred_element_type=jnp.float32)
    o_ref[...] = pt[:, None, :] + pi[None, :, :] + b_ref[...][None, :, :]


def _cross_table(emb_test, w_test, emb_inter, w_inter, b_proj):
    vt = emb_test.shape[0]
    out = pl.pallas_call(
        _cross_body,
        in_specs=[
            pl.BlockSpec((vt, D), lambda: (0, 0)),
            pl.BlockSpec((D, HD), lambda: (0, 0)),
            pl.BlockSpec((3, D), lambda: (0, 0)),
            pl.BlockSpec((D, HD), lambda: (0, 0)),
            pl.BlockSpec((1, HD), lambda: (0, 0)),
        ],
        out_specs=pl.BlockSpec((vt, 3, HD), lambda: (0, 0, 0)),
        out_shape=jax.ShapeDtypeStruct((vt, 3, HD), jnp.float32),
    )(emb_test, w_test, emb_inter, w_inter, b_proj.reshape(1, HD))
    return out.reshape(vt * 3, HD)


def _sc_gather_sum(p_ti, p_quest, p_tag, it, ii, iq, ig, half):
    mesh = plsc.VectorSubcoreMesh(core_axis_name="c", subcore_axis_name="s")
    idx_t = pltpu.VMEM((CHUNK,), jnp.int32)
    rows_t = pltpu.VMEM((CHUNK, HD), jnp.float32)
    sem = pltpu.SemaphoreType.DMA
    off = half * NH

    @functools.partial(
        pl.kernel,
        mesh=mesh,
        out_type=jax.ShapeDtypeStruct((NH, HD), jnp.float32),
        scratch_types=[idx_t] * 8 + [rows_t] * 2 + [sem] * 8,
    )
    def k(ti_h, quest_h, tag_h, it_h, ii_h, iq_h, ig_h, o_h,
          ivt0, ivt1, ivi0, ivi1, ivq0, ivq1, ivg0, ivg1, r0, r1,
          si0, si1, sg0, sg1, sa0, sa1, so0, so1):
        ivt, ivi, ivq, ivg = (ivt0, ivt1), (ivi0, ivi1), (ivq0, ivq1), (ivg0, ivg1)
        r = (r0, r1)
        s_idx, s_g, s_add, s_out = (si0, si1), (sg0, sg1), (sa0, sa1), (so0, so1)

        wid = lax.axis_index("s") * NC + lax.axis_index("c")
        w_base = wid * PER_W

        def base(i):
            return pl.multiple_of(w_base + i * CHUNK, CHUNK)

        def issue_idx(i):
            j = i % 2
            sl = pl.ds(off + base(i), CHUNK)
            return [
                pltpu.async_copy(it_h.at[sl], ivt[j], s_idx[j]),
                pltpu.async_copy(ii_h.at[sl], ivi[j], s_idx[j]),
                pltpu.async_copy(iq_h.at[sl], ivq[j], s_idx[j]),
                pltpu.async_copy(ig_h.at[sl], ivg[j], s_idx[j]),
            ]

        idx_d = {0: issue_idx(0)}
        add_d = {}
        out_d = {}
        for i in range(STEPS):
            j = i % 2
            for d in idx_d.pop(i):
                d.wait()

            def fuse(kk, carry, _j=j):
                sl = pl.ds(kk * LANES, LANES)
                ivt[_j][sl] = ivt[_j][sl] * 3 + ivi[_j][sl]
                return carry

            lax.fori_loop(0, CHUNK // LANES, fuse, 0)
            if i >= 2:
                out_d.pop(i - 2).wait()
            g = pltpu.async_copy(ti_h.at[ivt[j]], r[j], s_g[j])
            if i >= 1:
                for d in add_d.pop(i - 1):
                    d.wait()
                out_d[i - 1] = pltpu.async_copy(
                    r[1 - j], o_h.at[pl.ds(base(i - 1), CHUNK)], s_out[1 - j])
            if i + 1 < STEPS:
                idx_d[i + 1] = issue_idx(i + 1)
            g.wait()
            add_d[i] = [
                pltpu.async_copy(quest_h.at[ivq[j]], r[j], s_add[j], add=True),
                pltpu.async_copy(tag_h.at[ivg[j]], r[j], s_add[j], add=True),
            ]
        last = STEPS - 1
        jl = last % 2
        for d in add_d.pop(last):
            d.wait()
        out_d[last] = pltpu.async_copy(
            r[jl], o_h.at[pl.ds(base(last), CHUNK)], s_out[jl])
        out_d.pop(last - 1).wait()
        out_d.pop(last).wait()

    return k(p_ti, p_quest, p_tag, it, ii, iq, ig)


ROWS = 4096
GRID_H = NH // ROWS


def _ln_math(x, g, b):
    mu = jnp.mean(x, axis=-1, keepdims=True)
    xc = x - mu
    var = jnp.mean(xc * xc, axis=-1, keepdims=True)
    return xc * lax.rsqrt(var + 1e-6) * g + b


def _ln_body0(x_ref, g_ref, beta_ref, o_ref):
    o_ref[...] = _ln_math(x_ref[...], g_ref[...], beta_ref[...])


def _ln_body1(x_ref, g_ref, beta_ref, full_ref, o_ref):
    del full_ref
    o_ref[...] = _ln_math(x_ref[...], g_ref[...], beta_ref[...])


def _tc_epilogue(xs, ln_gamma, ln_beta):
    g2 = ln_gamma.reshape(1, HD)
    b2 = ln_beta.reshape(1, HD)
    y = pl.pallas_call(
        _ln_body0,
        grid=(GRID_H,),
        in_specs=[
            pl.BlockSpec((ROWS, HD), lambda i: (i, 0)),
            pl.BlockSpec((1, HD), lambda i: (0, 0)),
            pl.BlockSpec((1, HD), lambda i: (0, 0)),
        ],
        out_specs=pl.BlockSpec((ROWS, HD), lambda i: (i, 0)),
        out_shape=jax.ShapeDtypeStruct((N, HD), jnp.float32),
    )(xs[0], g2, b2)
    for h in range(1, NSLICE):
        y = pl.pallas_call(
            _ln_body1,
            grid=(GRID_H,),
            in_specs=[
                pl.BlockSpec((ROWS, HD), lambda i: (i, 0)),
                pl.BlockSpec((1, HD), lambda i: (0, 0)),
                pl.BlockSpec((1, HD), lambda i: (0, 0)),
                pl.BlockSpec(memory_space=pl.ANY),
            ],
            out_specs=pl.BlockSpec(
                (ROWS, HD), lambda i, _h=h: (i + _h * GRID_H, 0)),
            out_shape=jax.ShapeDtypeStruct((N, HD), jnp.float32),
            input_output_aliases={3: 0},
        )(xs[h], g2, b2, y)
    return y


def kernel(testId, assessmentItemID, KnowledgeTag, answerCode, mask,
           interaction, emb_interaction, emb_test, emb_question, emb_tag,
           W_proj, b_proj, ln_gamma, ln_beta):
    it = testId.reshape(N).astype(jnp.int32)
    ii = interaction.reshape(N).astype(jnp.int32)
    iq = assessmentItemID.reshape(N).astype(jnp.int32)
    ig = KnowledgeTag.reshape(N).astype(jnp.int32)
    p_ti = _cross_table(emb_test, W_proj[D:2 * D], emb_interaction,
                        W_proj[0:D], b_proj)
    p_quest = _project(emb_question.astype(jnp.bfloat16),
                       W_proj[2 * D:3 * D], bf16=True)
    p_tag = _project(emb_tag, W_proj[3 * D:4 * D])
    xs = [_sc_gather_sum(p_ti, p_quest, p_tag, it, ii, iq, ig, h)
          for h in range(NSLICE)]
    x = _tc_epilogue(xs, ln_gamma, ln_beta)
    return x.reshape(B, S, HD)
